# baseline (speedup 1.0000x reference)
import functools

import jax
import jax.numpy as jnp
from jax import lax
from jax.experimental import pallas as pl
from jax.experimental.pallas import tpu as pltpu
from jax.experimental.pallas import tpu_sc as plsc

NUM_GROUPS = 8
N_TOKENS = 4096
D_IN = 1024
D_OUT = 1024
TM = 256
NUM_TILES = N_TOKENS // TM
NUM_UNITS = NUM_TILES + (NUM_GROUPS - 1)
N_PAD = NUM_UNITS * TM

NUM_SC = 2
NUM_SUBCORES = 16
CHUNK = N_TOKENS // NUM_SUBCORES
HALF = CHUNK // NUM_SC

_MESH = plsc.VectorSubcoreMesh(core_axis_name="c", subcore_axis_name="s")


def _histogram_body(gi_hbm, cnts_hbm, gi_v, cnt_v):
    c = lax.axis_index("c")
    s = lax.axis_index("s")
    base = s * CHUNK
    pltpu.sync_copy(gi_hbm.at[pl.ds(base, CHUNK)], gi_v)
    lane = lax.broadcasted_iota(jnp.int32, (16,), 0)
    zero = jnp.zeros((16,), jnp.int32)
    counts = zero
    for v in range(CHUNK // 16):
        vec = gi_v[pl.ds(v * 16, 16)]
        for g in range(NUM_GROUPS):
            pc = jnp.sum((vec == g).astype(jnp.int32))
            counts = counts + jnp.where(lane == g, pc, zero)
    cnt_v[...] = counts

    @pl.when(c == 0)
    def _():
        pltpu.sync_copy(cnt_v, cnts_hbm.at[s])


@functools.partial(
    pl.kernel,
    out_type=jax.ShapeDtypeStruct((NUM_SUBCORES, 16), jnp.int32),
    mesh=_MESH,
    scratch_types=[
        pltpu.VMEM((CHUNK,), jnp.int32),
        pltpu.VMEM((16,), jnp.int32),
    ],
    compiler_params=pltpu.CompilerParams(needs_layout_passes=False),
)
def _histogram(gi_hbm, cnts_hbm, *rest):
    _histogram_body(gi_hbm, cnts_hbm, *rest)


def _dispatch_body(x_hbm, gi_hbm, cnts_hbm, xs_hbm, pos_hbm, gs_hbm,
                   gi_v, pos2_v, posf_v, rows0_v, rows1_v, call_v, gs_v,
                   rsem0, rsem1, ssem0, ssem1):
    c = lax.axis_index("c")
    s = lax.axis_index("s")
    base = s * CHUNK
    nb = HALF // 32
    bufs = [rows0_v, rows1_v]
    rsems = [rsem0, rsem1]
    ssems = [ssem0, ssem1]

    def read(i, bidx):
        r0 = base + (nb * c + i) * 32
        return pltpu.async_copy(x_hbm.at[pl.ds(r0, 32)], bufs[bidx], rsems[bidx])

    rd = [read(0, 0), read(1, 1)]

    pltpu.sync_copy(gi_hbm.at[pl.ds(base, CHUNK)], gi_v)
    pltpu.sync_copy(cnts_hbm, call_v)
    lane = lax.broadcasted_iota(jnp.int32, (16,), 0)
    zero = jnp.zeros((16,), jnp.int32)

    total = zero
    prefix = zero
    for w2 in range(NUM_SUBCORES):
        row = call_v[w2]
        total = total + row
        before = jnp.full((16,), w2, jnp.int32) < s
        prefix = prefix + jnp.where(before, row, zero)
    padded = ((total + (TM - 1)) // TM) * TM
    excl = jnp.cumsum(padded) - padded
    basev = excl + prefix

    @pl.when((c == 0) & (s == 0))
    def _():
        gs_v[...] = excl
        pltpu.sync_copy(gs_v, gs_hbm)

    run = [None] * NUM_GROUPS
    for g in range(NUM_GROUPS):
        run[g] = jnp.broadcast_to(jnp.sum(jnp.where(lane == g, basev, zero)), (16,))
    for v in range(CHUNK // 16):
        vec = gi_v[pl.ds(v * 16, 16)]
        posvec = zero
        for g in range(NUM_GROUPS):
            m = vec == g
            r = jnp.cumsum(m.astype(jnp.int32))
            posvec = jnp.where(m, run[g] + r - 1, posvec)
            run[g] = run[g] + jnp.sum(m.astype(jnp.int32))
        posf_v[pl.ds(v * 16, 16)] = posvec
        pos2_v[v // 2, pl.ds((v % 2) * 16, 16)] = posvec

    @pl.when(c == 0)
    def _():
        pltpu.sync_copy(posf_v, pos_hbm.at[pl.ds(base, CHUNK)])

    def scat(i, bidx):
        return pltpu.async_copy(
            bufs[bidx], xs_hbm.at[pos2_v.at[nb * c + i]], ssems[bidx]
        )

    sc = [None, None]
    for i in range(nb):
        bidx = i % 2
        rd[bidx].wait()
        sc[bidx] = scat(i, bidx)
        if i + 2 < nb:
            sc[bidx].wait()
            rd[bidx] = read(i + 2, bidx)
    sc[0].wait()
    sc[1].wait()


@functools.partial(
    pl.kernel,
    out_type=(
        jax.ShapeDtypeStruct((N_PAD, D_IN), jnp.float32),
        jax.ShapeDtypeStruct((N_TOKENS,), jnp.int32),
        jax.ShapeDtypeStruct((16,), jnp.int32),
    ),
    mesh=_MESH,
    scratch_types=[
        pltpu.VMEM((CHUNK,), jnp.int32),
        pltpu.VMEM((CHUNK // 32, 32), jnp.int32),
        pltpu.VMEM((CHUNK,), jnp.int32),
        pltpu.VMEM((32, D_IN), jnp.float32),
        pltpu.VMEM((32, D_IN), jnp.float32),
        pltpu.VMEM((NUM_SUBCORES, 16), jnp.int32),
        pltpu.VMEM((16,), jnp.int32),
        pltpu.SemaphoreType.DMA,
        pltpu.SemaphoreType.DMA,
        pltpu.SemaphoreType.DMA,
        pltpu.SemaphoreType.DMA,
    ],
    compiler_params=pltpu.CompilerParams(needs_layout_passes=False),
)
def _dispatch(x_hbm, gi_hbm, cnts_hbm, xs_hbm, pos_hbm, gs_hbm, *rest):
    _dispatch_body(x_hbm, gi_hbm, cnts_hbm, xs_hbm, pos_hbm, gs_hbm, *rest)


def _unsort_body(os_hbm, pos_hbm, out_hbm, posb_v, rows0_v, rows1_v,
                 gsem0, gsem1, wsem0, wsem1):
    c = lax.axis_index("c")
    s = lax.axis_index("s")
    wid = s * NUM_SC + c
    base = wid * (N_TOKENS // (NUM_SC * NUM_SUBCORES))
    pltpu.sync_copy(pos_hbm.at[pl.ds(base, 128)], posb_v)
    bufs = [rows0_v, rows1_v]
    gsems = [gsem0, gsem1]
    wsems = [wsem0, wsem1]

    def gath(i, bidx):
        return pltpu.async_copy(
            os_hbm.at[posb_v.at[pl.ds(i * 32, 32)]], bufs[bidx], gsems[bidx]
        )

    def wlin(i, bidx):
        return pltpu.async_copy(
            bufs[bidx], out_hbm.at[pl.ds(base + i * 32, 32)], wsems[bidx]
        )

    gd = [gath(0, 0), gath(1, 1)]
    wr = [None, None]
    for i in range(4):
        bidx = i % 2
        gd[bidx].wait()
        wr[bidx] = wlin(i, bidx)
        if i + 2 < 4:
            wr[bidx].wait()
            gd[bidx] = gath(i + 2, bidx)
    wr[0].wait()
    wr[1].wait()


@functools.partial(
    pl.kernel,
    out_type=jax.ShapeDtypeStruct((N_TOKENS, D_OUT), jnp.float32),
    mesh=_MESH,
    scratch_types=[
        pltpu.VMEM((128,), jnp.int32),
        pltpu.VMEM((32, D_OUT), jnp.float32),
        pltpu.VMEM((32, D_OUT), jnp.float32),
        pltpu.SemaphoreType.DMA,
        pltpu.SemaphoreType.DMA,
        pltpu.SemaphoreType.DMA,
        pltpu.SemaphoreType.DMA,
    ],
    compiler_params=pltpu.CompilerParams(needs_layout_passes=False),
)
def _unsort(os_hbm, pos_hbm, out_hbm, *rest):
    _unsort_body(os_hbm, pos_hbm, out_hbm, *rest)


def _gemm_body(starts_s, ends_s, tiles_s, groups_s, x_ref, w_ref, b_ref, o_ref):
    u = pl.program_id(0)
    lo = starts_s[u]
    hi = ends_s[u]
    t = tiles_s[u]

    del t

    @pl.when(lo < hi)
    def _():
        acc = jnp.dot(x_ref[...], w_ref[0], preferred_element_type=jnp.float32)
        o_ref[...] = acc + b_ref[0]


def _grouped_gemm(x_sorted, w, b, starts, ends, tiles, groups):
    grid_spec = pltpu.PrefetchScalarGridSpec(
        num_scalar_prefetch=4,
        grid=(NUM_UNITS,),
        in_specs=[
            pl.BlockSpec((TM, D_IN), lambda u, s, e, t, g: (t[u], 0)),
            pl.BlockSpec((1, D_IN, D_OUT), lambda u, s, e, t, g: (g[u], 0, 0)),
            pl.BlockSpec((1, 1, D_OUT), lambda u, s, e, t, g: (g[u], 0, 0)),
        ],
        out_specs=pl.BlockSpec((TM, D_OUT), lambda u, s, e, t, g: (t[u], 0)),
    )
    return pl.pallas_call(
        _gemm_body,
        grid_spec=grid_spec,
        out_shape=jax.ShapeDtypeStruct((N_PAD, D_OUT), jnp.float32),
        compiler_params=pltpu.CompilerParams(
            dimension_semantics=("arbitrary",),
        ),
    )(starts, ends, tiles, groups, x_sorted, w, b.reshape(NUM_GROUPS, 1, D_OUT))


def kernel(inputs, group_indices, w, b):
    gi = group_indices.astype(jnp.int32)
    oh = (gi.reshape(NUM_SUBCORES, CHUNK, 1)
          == jnp.arange(NUM_GROUPS, dtype=jnp.int32)).astype(jnp.int32)
    cnts = jnp.pad(jnp.sum(oh, axis=1), ((0, 0), (0, 16 - NUM_GROUPS)))
    x_sorted, pos, _ = _dispatch(inputs, gi, cnts)

    total = jnp.sum(cnts, axis=0)[:NUM_GROUPS]
    padded = ((total + (TM - 1)) // TM) * TM
    gs_pad = jnp.concatenate(
        [jnp.zeros((1,), jnp.int32), jnp.cumsum(padded)[: NUM_GROUPS - 1]]
    ).astype(jnp.int32)
    group_end = gs_pad + total

    starts = jnp.arange(NUM_UNITS, dtype=jnp.int32) * TM
    tiles = jnp.arange(NUM_UNITS, dtype=jnp.int32)
    groups = (
        jnp.searchsorted(gs_pad, starts, side="right").astype(jnp.int32) - 1
    )
    ends = jnp.minimum(starts + TM, group_end[groups])
    ends = jnp.maximum(ends, starts)

    out_sorted = _grouped_gemm(x_sorted, w, b, starts, ends, tiles, groups)
    return _unsort(out_sorted, pos)


# speedup vs baseline: 1.0234x; 1.0234x over previous
import functools

import jax
import jax.numpy as jnp
from jax import lax
from jax.experimental import pallas as pl
from jax.experimental.pallas import tpu as pltpu
from jax.experimental.pallas import tpu_sc as plsc

NUM_GROUPS = 8
N_TOKENS = 4096
D_IN = 1024
D_OUT = 1024
TM = 256
NUM_TILES = N_TOKENS // TM
NUM_UNITS = NUM_TILES + (NUM_GROUPS - 1)
N_PAD = NUM_UNITS * TM

NUM_SC = 2
NUM_SUBCORES = 16
CHUNK = N_TOKENS // NUM_SUBCORES
HALF = CHUNK // NUM_SC

_MESH = plsc.VectorSubcoreMesh(core_axis_name="c", subcore_axis_name="s")


def _histogram_body(gi_hbm, cnts_hbm, gi_v, cnt_v):
    c = lax.axis_index("c")
    s = lax.axis_index("s")
    base = s * CHUNK
    pltpu.sync_copy(gi_hbm.at[pl.ds(base, CHUNK)], gi_v)
    lane = lax.broadcasted_iota(jnp.int32, (16,), 0)
    zero = jnp.zeros((16,), jnp.int32)
    counts = zero
    for v in range(CHUNK // 16):
        vec = gi_v[pl.ds(v * 16, 16)]
        for g in range(NUM_GROUPS):
            pc = jnp.sum((vec == g).astype(jnp.int32))
            counts = counts + jnp.where(lane == g, pc, zero)
    cnt_v[...] = counts

    @pl.when(c == 0)
    def _():
        pltpu.sync_copy(cnt_v, cnts_hbm.at[s])


@functools.partial(
    pl.kernel,
    out_type=jax.ShapeDtypeStruct((NUM_SUBCORES, 16), jnp.int32),
    mesh=_MESH,
    scratch_types=[
        pltpu.VMEM((CHUNK,), jnp.int32),
        pltpu.VMEM((16,), jnp.int32),
    ],
    compiler_params=pltpu.CompilerParams(needs_layout_passes=False),
)
def _histogram(gi_hbm, cnts_hbm, *rest):
    _histogram_body(gi_hbm, cnts_hbm, *rest)


def _dispatch_body(x_hbm, gi_hbm, cnts_hbm, xs_hbm, pos_hbm, gs_hbm,
                   gi_v, pos2_v, posf_v, rows0_v, rows1_v, call_v, gs_v,
                   rsem0, rsem1, ssem0, ssem1):
    c = lax.axis_index("c")
    s = lax.axis_index("s")
    base = s * CHUNK
    nb = HALF // 32
    bufs = [rows0_v, rows1_v]
    rsems = [rsem0, rsem1]
    ssems = [ssem0, ssem1]

    def read(i, bidx):
        r0 = base + (nb * c + i) * 32
        return pltpu.async_copy(x_hbm.at[pl.ds(r0, 32)], bufs[bidx], rsems[bidx])

    rd = [read(0, 0), read(1, 1)]

    pltpu.sync_copy(gi_hbm.at[pl.ds(base, CHUNK)], gi_v)
    pltpu.sync_copy(cnts_hbm, call_v)
    lane = lax.broadcasted_iota(jnp.int32, (16,), 0)
    zero = jnp.zeros((16,), jnp.int32)

    total = zero
    prefix = zero
    for w2 in range(NUM_SUBCORES):
        row = call_v[w2]
        total = total + row
        before = jnp.full((16,), w2, jnp.int32) < s
        prefix = prefix + jnp.where(before, row, zero)
    padded = ((total + (TM - 1)) // TM) * TM
    excl = jnp.cumsum(padded) - padded
    basev = excl + prefix

    @pl.when((c == 0) & (s == 0))
    def _():
        gs_v[...] = excl
        pltpu.sync_copy(gs_v, gs_hbm)

    run = [None] * NUM_GROUPS
    for g in range(NUM_GROUPS):
        run[g] = jnp.broadcast_to(jnp.sum(jnp.where(lane == g, basev, zero)), (16,))
    for v in range(CHUNK // 16):
        vec = gi_v[pl.ds(v * 16, 16)]
        posvec = zero
        for g in range(NUM_GROUPS):
            m = vec == g
            r = jnp.cumsum(m.astype(jnp.int32))
            posvec = jnp.where(m, run[g] + r - 1, posvec)
            run[g] = run[g] + jnp.sum(m.astype(jnp.int32))
        posf_v[pl.ds(v * 16, 16)] = posvec
        pos2_v[v // 2, pl.ds((v % 2) * 16, 16)] = posvec

    @pl.when(c == 0)
    def _():
        pltpu.sync_copy(posf_v, pos_hbm.at[pl.ds(base, CHUNK)])

    def scat(i, bidx):
        return pltpu.async_copy(
            bufs[bidx], xs_hbm.at[pos2_v.at[nb * c + i]], ssems[bidx]
        )

    sc = [None, None]
    for i in range(nb):
        bidx = i % 2
        rd[bidx].wait()
        sc[bidx] = scat(i, bidx)
        if i + 2 < nb:
            sc[bidx].wait()
            rd[bidx] = read(i + 2, bidx)
    sc[0].wait()
    sc[1].wait()


@functools.partial(
    pl.kernel,
    out_type=(
        jax.ShapeDtypeStruct((N_PAD, D_IN), jnp.float32),
        jax.ShapeDtypeStruct((N_TOKENS,), jnp.int32),
        jax.ShapeDtypeStruct((16,), jnp.int32),
    ),
    mesh=_MESH,
    scratch_types=[
        pltpu.VMEM((CHUNK,), jnp.int32),
        pltpu.VMEM((CHUNK // 32, 32), jnp.int32),
        pltpu.VMEM((CHUNK,), jnp.int32),
        pltpu.VMEM((32, D_IN), jnp.float32),
        pltpu.VMEM((32, D_IN), jnp.float32),
        pltpu.VMEM((NUM_SUBCORES, 16), jnp.int32),
        pltpu.VMEM((16,), jnp.int32),
        pltpu.SemaphoreType.DMA,
        pltpu.SemaphoreType.DMA,
        pltpu.SemaphoreType.DMA,
        pltpu.SemaphoreType.DMA,
    ],
    compiler_params=pltpu.CompilerParams(needs_layout_passes=False),
)
def _dispatch(x_hbm, gi_hbm, cnts_hbm, xs_hbm, pos_hbm, gs_hbm, *rest):
    _dispatch_body(x_hbm, gi_hbm, cnts_hbm, xs_hbm, pos_hbm, gs_hbm, *rest)


def _unsort_body(os_hbm, pos_hbm, out_hbm, posb_v, rows0_v, rows1_v,
                 gsem0, gsem1, wsem0, wsem1):
    c = lax.axis_index("c")
    s = lax.axis_index("s")
    wid = s * NUM_SC + c
    base = wid * (N_TOKENS // (NUM_SC * NUM_SUBCORES))
    pltpu.sync_copy(pos_hbm.at[pl.ds(base, 128)], posb_v)
    bufs = [rows0_v, rows1_v]
    gsems = [gsem0, gsem1]
    wsems = [wsem0, wsem1]

    def gath(i, bidx):
        return pltpu.async_copy(
            os_hbm.at[posb_v.at[pl.ds(i * 32, 32)]], bufs[bidx], gsems[bidx]
        )

    def wlin(i, bidx):
        return pltpu.async_copy(
            bufs[bidx], out_hbm.at[pl.ds(base + i * 32, 32)], wsems[bidx]
        )

    gd = [gath(0, 0), gath(1, 1)]
    wr = [None, None]
    for i in range(4):
        bidx = i % 2
        gd[bidx].wait()
        wr[bidx] = wlin(i, bidx)
        if i + 2 < 4:
            wr[bidx].wait()
            gd[bidx] = gath(i + 2, bidx)
    wr[0].wait()
    wr[1].wait()


@functools.partial(
    pl.kernel,
    out_type=jax.ShapeDtypeStruct((N_TOKENS, D_OUT), jnp.float32),
    mesh=_MESH,
    scratch_types=[
        pltpu.VMEM((128,), jnp.int32),
        pltpu.VMEM((32, D_OUT), jnp.float32),
        pltpu.VMEM((32, D_OUT), jnp.float32),
        pltpu.SemaphoreType.DMA,
        pltpu.SemaphoreType.DMA,
        pltpu.SemaphoreType.DMA,
        pltpu.SemaphoreType.DMA,
    ],
    compiler_params=pltpu.CompilerParams(needs_layout_passes=False),
)
def _unsort(os_hbm, pos_hbm, out_hbm, *rest):
    _unsort_body(os_hbm, pos_hbm, out_hbm, *rest)


def _gemm_body(starts_s, ends_s, tiles_s, groups_s, x_ref, w_ref, b_ref, o_ref):
    u = pl.program_id(0)
    lo = starts_s[u]
    hi = ends_s[u]
    t = tiles_s[u]

    del t

    @pl.when(lo < hi)
    def _():
        acc = jnp.dot(x_ref[...], w_ref[0], preferred_element_type=jnp.float32)
        o_ref[...] = acc + b_ref[0]


def _grouped_gemm(x_sorted, w, b, starts, ends, tiles, groups):
    grid_spec = pltpu.PrefetchScalarGridSpec(
        num_scalar_prefetch=4,
        grid=(NUM_UNITS,),
        in_specs=[
            pl.BlockSpec((TM, D_IN), lambda u, s, e, t, g: (t[u], 0)),
            pl.BlockSpec((1, D_IN, D_OUT), lambda u, s, e, t, g: (g[u], 0, 0)),
            pl.BlockSpec((1, 1, D_OUT), lambda u, s, e, t, g: (g[u], 0, 0)),
        ],
        out_specs=pl.BlockSpec((TM, D_OUT), lambda u, s, e, t, g: (t[u], 0)),
    )
    return pl.pallas_call(
        _gemm_body,
        grid_spec=grid_spec,
        out_shape=jax.ShapeDtypeStruct((N_PAD, D_OUT), jnp.float32),
        compiler_params=pltpu.CompilerParams(
            dimension_semantics=("arbitrary",),
        ),
    )(starts, ends, tiles, groups, x_sorted, w, b.reshape(NUM_GROUPS, 1, D_OUT))


def kernel(inputs, group_indices, w, b):
    gi = group_indices.astype(jnp.int32)
    oh = (gi.reshape(NUM_SUBCORES, CHUNK, 1)
          == jnp.arange(NUM_GROUPS, dtype=jnp.int32)).astype(jnp.int32)
    cnts = jnp.pad(jnp.sum(oh, axis=1), ((0, 0), (0, 16 - NUM_GROUPS)))
    x_sorted, pos, _ = _dispatch(inputs, gi, cnts)

    total = jnp.sum(cnts, axis=0)[:NUM_GROUPS]
    padded = ((total + (TM - 1)) // TM) * TM
    gs_pad = jnp.concatenate(
        [jnp.zeros((1,), jnp.int32), jnp.cumsum(padded)[: NUM_GROUPS - 1]]
    ).astype(jnp.int32)
    group_end = gs_pad + total

    used = jnp.sum(padded) // TM
    starts = jnp.arange(NUM_UNITS, dtype=jnp.int32) * TM
    tiles = jnp.minimum(jnp.arange(NUM_UNITS, dtype=jnp.int32), used - 1)
    tstarts = tiles * TM
    groups = (
        jnp.searchsorted(gs_pad, tstarts, side="right").astype(jnp.int32) - 1
    )
    ends = jnp.minimum(tstarts + TM, group_end[groups])
    ends = jnp.maximum(ends, starts)

    out_sorted = _grouped_gemm(x_sorted, w, b, starts, ends, tiles, groups)
    return _unsort(out_sorted, pos)


# speedup vs baseline: 1.0428x; 1.0189x over previous
import functools

import jax
import jax.numpy as jnp
from jax import lax
from jax.experimental import pallas as pl
from jax.experimental.pallas import tpu as pltpu
from jax.experimental.pallas import tpu_sc as plsc

NUM_GROUPS = 8
N_TOKENS = 4096
D_IN = 1024
D_OUT = 1024
TM = 256
NUM_TILES = N_TOKENS // TM
NUM_UNITS = NUM_TILES + (NUM_GROUPS - 1)

NUM_SC = 2
NUM_SUBCORES = 16
CHUNK = N_TOKENS // NUM_SUBCORES
HALF = CHUNK // NUM_SC

_MESH = plsc.VectorSubcoreMesh(core_axis_name="c", subcore_axis_name="s")


def _histogram_body(gi_hbm, cnts_hbm, gi_v, cnt_v):
    c = lax.axis_index("c")
    s = lax.axis_index("s")
    base = s * CHUNK
    pltpu.sync_copy(gi_hbm.at[pl.ds(base, CHUNK)], gi_v)
    lane = lax.broadcasted_iota(jnp.int32, (16,), 0)
    zero = jnp.zeros((16,), jnp.int32)
    counts = zero
    for v in range(CHUNK // 16):
        vec = gi_v[pl.ds(v * 16, 16)]
        for g in range(NUM_GROUPS):
            pc = jnp.sum((vec == g).astype(jnp.int32))
            counts = counts + jnp.where(lane == g, pc, zero)
    cnt_v[...] = counts

    @pl.when(c == 0)
    def _():
        pltpu.sync_copy(cnt_v, cnts_hbm.at[s])


@functools.partial(
    pl.kernel,
    out_type=jax.ShapeDtypeStruct((NUM_SUBCORES, 16), jnp.int32),
    mesh=_MESH,
    scratch_types=[
        pltpu.VMEM((CHUNK,), jnp.int32),
        pltpu.VMEM((16,), jnp.int32),
    ],
    compiler_params=pltpu.CompilerParams(needs_layout_passes=False),
)
def _histogram(gi_hbm, cnts_hbm, *rest):
    _histogram_body(gi_hbm, cnts_hbm, *rest)


def _dispatch_body(x_hbm, gi_hbm, cnts_hbm, xs_hbm, pos_hbm, gs_hbm,
                   gi_v, pos2_v, posf_v, rows0_v, rows1_v, call_v, gs_v,
                   rsem0, rsem1, ssem0, ssem1):
    c = lax.axis_index("c")
    s = lax.axis_index("s")
    base = s * CHUNK
    nb = HALF // 32
    bufs = [rows0_v, rows1_v]
    rsems = [rsem0, rsem1]
    ssems = [ssem0, ssem1]

    def read(i, bidx):
        r0 = base + (nb * c + i) * 32
        return pltpu.async_copy(x_hbm.at[pl.ds(r0, 32)], bufs[bidx], rsems[bidx])

    rd = [read(0, 0), read(1, 1)]

    pltpu.sync_copy(gi_hbm.at[pl.ds(base, CHUNK)], gi_v)
    pltpu.sync_copy(cnts_hbm, call_v)
    lane = lax.broadcasted_iota(jnp.int32, (16,), 0)
    zero = jnp.zeros((16,), jnp.int32)

    total = zero
    prefix = zero
    for w2 in range(NUM_SUBCORES):
        row = call_v[w2]
        total = total + row
        before = jnp.full((16,), w2, jnp.int32) < s
        prefix = prefix + jnp.where(before, row, zero)
    excl = jnp.cumsum(total) - total
    basev = excl + prefix

    @pl.when((c == 0) & (s == 0))
    def _():
        gs_v[...] = excl
        pltpu.sync_copy(gs_v, gs_hbm)

    run = [None] * NUM_GROUPS
    for g in range(NUM_GROUPS):
        run[g] = jnp.broadcast_to(jnp.sum(jnp.where(lane == g, basev, zero)), (16,))
    for v in range(CHUNK // 16):
        vec = gi_v[pl.ds(v * 16, 16)]
        posvec = zero
        for g in range(NUM_GROUPS):
            m = vec == g
            r = jnp.cumsum(m.astype(jnp.int32))
            posvec = jnp.where(m, run[g] + r - 1, posvec)
            run[g] = run[g] + jnp.sum(m.astype(jnp.int32))
        posf_v[pl.ds(v * 16, 16)] = posvec
        pos2_v[v // 2, pl.ds((v % 2) * 16, 16)] = posvec

    @pl.when(c == 0)
    def _():
        pltpu.sync_copy(posf_v, pos_hbm.at[pl.ds(base, CHUNK)])

    def scat(i, bidx):
        return pltpu.async_copy(
            bufs[bidx], xs_hbm.at[pos2_v.at[nb * c + i]], ssems[bidx]
        )

    sc = [None, None]
    for i in range(nb):
        bidx = i % 2
        rd[bidx].wait()
        sc[bidx] = scat(i, bidx)
        if i + 2 < nb:
            sc[bidx].wait()
            rd[bidx] = read(i + 2, bidx)
    sc[0].wait()
    sc[1].wait()


@functools.partial(
    pl.kernel,
    out_type=(
        jax.ShapeDtypeStruct((N_TOKENS, D_IN), jnp.float32),
        jax.ShapeDtypeStruct((N_TOKENS,), jnp.int32),
        jax.ShapeDtypeStruct((16,), jnp.int32),
    ),
    mesh=_MESH,
    scratch_types=[
        pltpu.VMEM((CHUNK,), jnp.int32),
        pltpu.VMEM((CHUNK // 32, 32), jnp.int32),
        pltpu.VMEM((CHUNK,), jnp.int32),
        pltpu.VMEM((32, D_IN), jnp.float32),
        pltpu.VMEM((32, D_IN), jnp.float32),
        pltpu.VMEM((NUM_SUBCORES, 16), jnp.int32),
        pltpu.VMEM((16,), jnp.int32),
        pltpu.SemaphoreType.DMA,
        pltpu.SemaphoreType.DMA,
        pltpu.SemaphoreType.DMA,
        pltpu.SemaphoreType.DMA,
    ],
    compiler_params=pltpu.CompilerParams(needs_layout_passes=False),
)
def _dispatch(x_hbm, gi_hbm, cnts_hbm, xs_hbm, pos_hbm, gs_hbm, *rest):
    _dispatch_body(x_hbm, gi_hbm, cnts_hbm, xs_hbm, pos_hbm, gs_hbm, *rest)


def _unsort_body(os_hbm, pos_hbm, out_hbm, posb_v, rows0_v, rows1_v,
                 gsem0, gsem1, wsem0, wsem1):
    c = lax.axis_index("c")
    s = lax.axis_index("s")
    wid = s * NUM_SC + c
    base = wid * (N_TOKENS // (NUM_SC * NUM_SUBCORES))
    pltpu.sync_copy(pos_hbm.at[pl.ds(base, 128)], posb_v)
    bufs = [rows0_v, rows1_v]
    gsems = [gsem0, gsem1]
    wsems = [wsem0, wsem1]

    def gath(i, bidx):
        return pltpu.async_copy(
            os_hbm.at[posb_v.at[pl.ds(i * 32, 32)]], bufs[bidx], gsems[bidx]
        )

    def wlin(i, bidx):
        return pltpu.async_copy(
            bufs[bidx], out_hbm.at[pl.ds(base + i * 32, 32)], wsems[bidx]
        )

    gd = [gath(0, 0), gath(1, 1)]
    wr = [None, None]
    for i in range(4):
        bidx = i % 2
        gd[bidx].wait()
        wr[bidx] = wlin(i, bidx)
        if i + 2 < 4:
            wr[bidx].wait()
            gd[bidx] = gath(i + 2, bidx)
    wr[0].wait()
    wr[1].wait()


@functools.partial(
    pl.kernel,
    out_type=jax.ShapeDtypeStruct((N_TOKENS, D_OUT), jnp.float32),
    mesh=_MESH,
    scratch_types=[
        pltpu.VMEM((128,), jnp.int32),
        pltpu.VMEM((32, D_OUT), jnp.float32),
        pltpu.VMEM((32, D_OUT), jnp.float32),
        pltpu.SemaphoreType.DMA,
        pltpu.SemaphoreType.DMA,
        pltpu.SemaphoreType.DMA,
        pltpu.SemaphoreType.DMA,
    ],
    compiler_params=pltpu.CompilerParams(needs_layout_passes=False),
)
def _unsort(os_hbm, pos_hbm, out_hbm, *rest):
    _unsort_body(os_hbm, pos_hbm, out_hbm, *rest)


def _gemm_body(starts_s, ends_s, tiles_s, groups_s, x_ref, w_ref, b_ref, o_ref):
    u = pl.program_id(0)
    lo = starts_s[u]
    hi = ends_s[u]
    t = tiles_s[u]

    @pl.when(lo < hi)
    def _():
        acc = jnp.dot(x_ref[...], w_ref[0], preferred_element_type=jnp.float32)
        acc = acc + b_ref[0]
        ridx = jax.lax.broadcasted_iota(jnp.int32, (TM, 1), 0)
        bs = t * TM
        mask = (ridx >= lo - bs) & (ridx < hi - bs)
        o_ref[...] = jnp.where(mask, acc, o_ref[...])


def _grouped_gemm(x_sorted, w, b, starts, ends, tiles, groups):
    grid_spec = pltpu.PrefetchScalarGridSpec(
        num_scalar_prefetch=4,
        grid=(NUM_UNITS,),
        in_specs=[
            pl.BlockSpec((TM, D_IN), lambda u, s, e, t, g: (t[u], 0)),
            pl.BlockSpec((1, D_IN, D_OUT), lambda u, s, e, t, g: (g[u], 0, 0)),
            pl.BlockSpec((1, 1, D_OUT), lambda u, s, e, t, g: (g[u], 0, 0)),
        ],
        out_specs=pl.BlockSpec((TM, D_OUT), lambda u, s, e, t, g: (t[u], 0)),
    )
    return pl.pallas_call(
        _gemm_body,
        grid_spec=grid_spec,
        out_shape=jax.ShapeDtypeStruct((N_TOKENS, D_OUT), jnp.float32),
        compiler_params=pltpu.CompilerParams(
            dimension_semantics=("arbitrary",),
        ),
    )(starts, ends, tiles, groups, x_sorted, w, b.reshape(NUM_GROUPS, 1, D_OUT))


def kernel(inputs, group_indices, w, b):
    gi = group_indices.astype(jnp.int32)
    oh = (gi.reshape(NUM_SUBCORES, CHUNK, 1)
          == jnp.arange(NUM_GROUPS, dtype=jnp.int32)).astype(jnp.int32)
    cnts = jnp.pad(jnp.sum(oh, axis=1), ((0, 0), (0, 16 - NUM_GROUPS)))
    x_sorted, pos, _ = _dispatch(inputs, gi, cnts)

    total = jnp.sum(cnts, axis=0)
    group_starts = jnp.concatenate(
        [jnp.zeros((1,), jnp.int32), jnp.cumsum(total)[: NUM_GROUPS - 1]]
    ).astype(jnp.int32)

    tile_starts = jnp.arange(NUM_TILES, dtype=jnp.int32) * TM
    starts = jnp.sort(jnp.concatenate([tile_starts, group_starts[1:]]))
    ends = jnp.concatenate([starts[1:], jnp.full((1,), N_TOKENS, jnp.int32)])
    tiles = jnp.minimum(starts, N_TOKENS - 1) // TM
    groups = (
        jnp.searchsorted(group_starts, starts, side="right").astype(jnp.int32) - 1
    )

    out_sorted = _grouped_gemm(x_sorted, w, b, starts, ends, tiles, groups)
    return _unsort(out_sorted, pos)
